# baseline (speedup 1.0000x reference)
import jax
import jax.numpy as jnp
from jax.experimental import pallas as pl

_B, _C, _H, _W = 16, 96, 224, 224
_ROWS = _B * _C
_COLS = _H * _W
_RB = 32


def _body(x_ref, o_ref):
    i = pl.program_id(0)
    rows = jax.lax.broadcasted_iota(jnp.int32, (_RB, 1), 0) + i * _RB
    mask = (rows % _C) < 2
    o_ref[...] = jnp.where(mask, jnp.float32(4.0), x_ref[...])


def kernel(x):
    x2 = x.reshape(_ROWS, _COLS)
    y2 = pl.pallas_call(
        _body,
        grid=(_ROWS // _RB,),
        in_specs=[pl.BlockSpec((_RB, _COLS), lambda i: (i, 0))],
        out_specs=pl.BlockSpec((_RB, _COLS), lambda i: (i, 0)),
        out_shape=jax.ShapeDtypeStruct((_ROWS, _COLS), x.dtype),
    )(x2)
    return y2.reshape(_B, _C, _H, _W)


# speedup vs baseline: 3.8273x; 3.8273x over previous
import jax
import jax.numpy as jnp
from jax.experimental import pallas as pl

_B, _C, _H, _W = 16, 96, 224, 224
_CB = 16


def _body(x_ref, o_ref):
    j = pl.program_id(1)

    @pl.when(j == 0)
    def _():
        c = jax.lax.broadcasted_iota(jnp.int32, (1, _CB, 1, 1), 1)
        o_ref[...] = jnp.where(c < 2, jnp.float32(4.0), x_ref[...])

    @pl.when(j != 0)
    def _():
        o_ref[...] = x_ref[...]


def kernel(x):
    return pl.pallas_call(
        _body,
        grid=(_B, _C // _CB),
        in_specs=[pl.BlockSpec((1, _CB, _H, _W), lambda i, j: (i, j, 0, 0))],
        out_specs=pl.BlockSpec((1, _CB, _H, _W), lambda i, j: (i, j, 0, 0)),
        out_shape=jax.ShapeDtypeStruct((_B, _C, _H, _W), x.dtype),
    )(x)
